# baseline (speedup 1.0000x reference)
import functools

import jax
import jax.numpy as jnp
from jax import lax
from jax.experimental import pallas as pl
from jax.experimental.pallas import tpu as pltpu
from jax.experimental.pallas import tpu_sc as plsc

_L = 16


def _build(R, N, NW):
    rows_per_w = R // NW
    NV = N // _L
    mesh = plsc.VectorSubcoreMesh(core_axis_name="c", subcore_axis_name="s")
    NC = mesh.num_cores

    @functools.partial(
        pl.kernel,
        out_type=jax.ShapeDtypeStruct((R, N), jnp.float32),
        mesh=mesh,
        scratch_types=[
            pltpu.VMEM((N,), jnp.float32),
            pltpu.VMEM((N + _L,), jnp.float32),
            pltpu.VMEM((256 * _L,), jnp.int32),
            pltpu.VMEM((256 * _L,), jnp.int32),
            pltpu.VMEM((_L,), jnp.int32),
        ],
        compiler_params=pltpu.CompilerParams(needs_layout_passes=False),
    )
    def run(scores_hbm, kk_hbm, out_hbm, row_v, cand_v, hist_v, suf_v,
            kk_v):
        wid = lax.axis_index("s") * NC + lax.axis_index("c")
        pltpu.sync_copy(kk_hbm, kk_v)
        kk0 = kk_v[...][0]
        lanes = lax.iota(jnp.int32, _L)
        ones = jnp.ones((_L,), jnp.int32)
        zeros = jnp.zeros((_L,), jnp.int32)

        def unrolled(n_iter, unroll, body):
            @plsc.parallel_loop(0, n_iter, unroll=unroll)
            def _loop(i):
                body(i)

        def key_of(v):
            b = plsc.bitcast(v, jnp.int32)
            return b ^ (lax.shift_right_arithmetic(b, 31)
                        | jnp.int32(-0x80000000))

        def clear_hist():
            def clr(i):
                hist_v[pl.ds(i * _L, _L)] = zeros

            unrolled(256, 8, clr)

        def suffix_and_search(kk, remap=False):
            if remap:
                def loc(b):
                    return jnp.where(b >= 128, b ^ 128, 255 - b)
            else:
                def loc(b):
                    return b

            @plsc.parallel_loop(0, 256, unroll=8, carry=zeros)
            def sfx(i, s):
                bb = loc(255 - i)
                s = s + hist_v[pl.ds(bb * _L, _L)]
                hist_v[pl.ds(bb * _L, _L)] = zeros
                suf_v[pl.ds((255 - i) * _L, _L)] = s
                return s

            def bsearch(_i, lohi):
                lo, hi = lohi
                m = lax.shift_right_logical(lo + hi, 1)
                c = jnp.sum(suf_v[pl.ds(m * _L, _L)])
                take = c >= kk
                return (jnp.where(take, m, lo), jnp.where(take, hi, m))

            B, _hi = lax.fori_loop(
                0, 8, bsearch, (jnp.int32(0), jnp.int32(256)))
            c_above = jnp.where(
                B < 255,
                jnp.sum(suf_v[pl.ds(jnp.minimum(B + 1, 255) * _L,
                                         _L)]),
                0)
            return B, kk - c_above

        GV = 64
        NG = NV // GV
        neg_inf = jnp.full((_L,), -jnp.inf, jnp.float32)

        def do_row(rr, _):
            row_idx = wid * rows_per_w + rr
            pltpu.sync_copy(scores_hbm.at[row_idx], row_v)

            @plsc.parallel_loop(0, NG)
            def gmax(g):
                def inner(j, acc):
                    base = (g * GV + j * 8) * _L
                    for u in range(8):
                        acc = jnp.maximum(
                            acc, row_v[pl.ds(base + u * _L, _L)])
                    return acc

                m = lax.fori_loop(0, GV // 8, inner, neg_inf)
                cand_v[pl.ds(g * _L, _L)] = m

            def histm(i):
                t = lax.shift_right_logical(
                    plsc.bitcast(cand_v[pl.ds(i * _L, _L)], jnp.int32), 24)
                plsc.addupdate_scatter(hist_v, [t * _L + lanes], ones)

            unrolled(NG, 8, histm)
            Bp, _r = suffix_and_search(kk0, remap=True)

            key_edge = lax.shift_left(Bp, 24)
            tau_bits = jnp.where(
                key_edge < 0, key_edge ^ jnp.int32(-0x80000000), ~key_edge)
            tau_v = jnp.broadcast_to(
                lax.bitcast_convert_type(tau_bits, jnp.float32), (_L,))

            @plsc.parallel_loop(0, NV // 8, unroll=2,
                                carry=jnp.zeros((_L,), jnp.int32))
            def comp8(i, offv):
                vs, masks, offs = [], [], []
                for u in range(8):
                    v = row_v[pl.ds((i * 8 + u) * _L, _L)]
                    mask = v >= tau_v
                    vs.append(v)
                    masks.append(mask)
                    offs.append(offv)
                    offv = offv + plsc.all_reduce_population_count(mask)
                for u in range(8):
                    plsc.store_compressed(
                        cand_v.at[pl.ds(offs[u][0], _L)], vs[u],
                        mask=masks[u])
                return offv

            n_cand = comp8[0]
            nv_c = lax.shift_right_logical(n_cand + (_L - 1), 4)

            prefix = jnp.int32(0)
            kk = kk0
            for shift in (24, 16, 8, 0):
                @plsc.parallel_loop(0, nv_c)
                def hist2(i):
                    key = key_of(cand_v[pl.ds(i * _L, _L)])
                    valid = (i * _L + lanes) < n_cand
                    if shift == 24:
                        mask = valid
                    else:
                        mask = valid & (
                            lax.shift_right_logical(key, shift + 8)
                            == prefix)
                    byte = lax.shift_right_logical(key, shift) & 255
                    plsc.addupdate_scatter(
                        hist_v, [byte * _L + lanes], ones, mask=mask)

                B, kk = suffix_and_search(kk)
                prefix = B if shift == 24 else (prefix * 256) | B

            bits = jnp.where(
                prefix < 0, prefix ^ jnp.int32(-0x80000000), ~prefix)
            thr = lax.bitcast_convert_type(bits, jnp.float32)
            thrv = jnp.broadcast_to(thr, (_L,))

            def scale(i):
                v = row_v[pl.ds(i * _L, _L)]
                row_v[pl.ds(i * _L, _L)] = jnp.where(v >= thrv, v * 10.0, v)

            unrolled(NV, 16, scale)
            pltpu.sync_copy(row_v, out_hbm.at[row_idx])
            return 0

        clear_hist()
        lax.fori_loop(0, rows_per_w, do_row, 0)

    return run


def kernel(scores, k):
    R, N = scores.shape
    info = plsc.get_sparse_core_info()
    NW = info.num_cores * info.num_subcores
    kk = jnp.full((_L,), jnp.asarray(k, jnp.int32) + 1, jnp.int32)
    return _build(R, N, NW)(scores, kk)


# speedup vs baseline: 1.1322x; 1.1322x over previous
import functools

import jax
import jax.numpy as jnp
from jax import lax
from jax.experimental import pallas as pl
from jax.experimental.pallas import tpu as pltpu
from jax.experimental.pallas import tpu_sc as plsc

_L = 16


def _build(R, N, NW):
    rows_per_w = R // NW
    NV = N // _L
    mesh = plsc.VectorSubcoreMesh(core_axis_name="c", subcore_axis_name="s")
    NC = mesh.num_cores

    @functools.partial(
        pl.kernel,
        out_type=jax.ShapeDtypeStruct((R, N), jnp.float32),
        mesh=mesh,
        scratch_types=[
            pltpu.VMEM((N,), jnp.float32),
            pltpu.VMEM((N + _L,), jnp.float32),
            pltpu.VMEM((256 * _L,), jnp.int32),
            pltpu.VMEM((_L,), jnp.int32),
        ],
        compiler_params=pltpu.CompilerParams(needs_layout_passes=False),
    )
    def run(scores_hbm, kk_hbm, out_hbm, row_v, cand_v, hist_v, kk_v):
        wid = lax.axis_index("s") * NC + lax.axis_index("c")
        pltpu.sync_copy(kk_hbm, kk_v)
        kk0 = kk_v[...][0]
        lanes = lax.iota(jnp.int32, _L)
        ones = jnp.ones((_L,), jnp.int32)
        zeros = jnp.zeros((_L,), jnp.int32)

        def unrolled(n_iter, unroll, body):
            @plsc.parallel_loop(0, n_iter, unroll=unroll)
            def _loop(i):
                body(i)

        def key_of(v):
            b = plsc.bitcast(v, jnp.int32)
            return b ^ (lax.shift_right_arithmetic(b, 31)
                        | jnp.int32(-0x80000000))

        def clear_hist():
            def clr(i):
                hist_v[pl.ds(i * _L, _L)] = zeros

            unrolled(256, 8, clr)

        def suffix_and_search(kk, remap=False):
            if remap:
                def loc(b):
                    return jnp.where(b >= 128, b ^ 128, 255 - b)
            else:
                def loc(b):
                    return b

            @plsc.parallel_loop(0, 256, unroll=8, carry=zeros)
            def sfx(i, s):
                bb = loc(255 - i)
                s = s + hist_v[pl.ds(bb * _L, _L)]
                hist_v[pl.ds(bb * _L, _L)] = s
                return s

            def bsearch(_i, lohi):
                lo, hi = lohi
                m = lax.shift_right_logical(lo + hi, 1)
                c = jnp.sum(hist_v[pl.ds(loc(m) * _L, _L)])
                take = c >= kk
                return (jnp.where(take, m, lo), jnp.where(take, hi, m))

            B, _hi = lax.fori_loop(
                0, 8, bsearch, (jnp.int32(0), jnp.int32(256)))
            c_above = jnp.where(
                B < 255,
                jnp.sum(hist_v[pl.ds(loc(jnp.minimum(B + 1, 255)) * _L,
                                     _L)]),
                0)
            return B, kk - c_above

        GV = 64
        NG = NV // GV
        neg_inf = jnp.full((_L,), -jnp.inf, jnp.float32)

        def do_row(rr, _):
            row_idx = wid * rows_per_w + rr
            pltpu.sync_copy(scores_hbm.at[row_idx], row_v)

            @plsc.parallel_loop(0, NG)
            def gmax(g):
                def inner(j, acc):
                    base = (g * GV + j * 8) * _L
                    for u in range(8):
                        acc = jnp.maximum(
                            acc, row_v[pl.ds(base + u * _L, _L)])
                    return acc

                m = lax.fori_loop(0, GV // 8, inner, neg_inf)
                cand_v[pl.ds(g * _L, _L)] = m

            clear_hist()

            def histm(i):
                t = lax.shift_right_logical(
                    plsc.bitcast(cand_v[pl.ds(i * _L, _L)], jnp.int32), 24)
                plsc.addupdate_scatter(hist_v, [t * _L + lanes], ones)

            unrolled(NG, 8, histm)
            Bp, _r = suffix_and_search(kk0, remap=True)

            key_edge = lax.shift_left(Bp, 24)
            tau_bits = jnp.where(
                key_edge < 0, key_edge ^ jnp.int32(-0x80000000), ~key_edge)
            tau_v = jnp.broadcast_to(
                lax.bitcast_convert_type(tau_bits, jnp.float32), (_L,))

            @plsc.parallel_loop(0, NV // 8, unroll=2,
                                carry=jnp.zeros((_L,), jnp.int32))
            def comp8(i, offv):
                vs, masks, offs = [], [], []
                for u in range(8):
                    v = row_v[pl.ds((i * 8 + u) * _L, _L)]
                    mask = v >= tau_v
                    vs.append(v)
                    masks.append(mask)
                    offs.append(offv)
                    offv = offv + plsc.all_reduce_population_count(mask)
                for u in range(8):
                    plsc.store_compressed(
                        cand_v.at[pl.ds(offs[u][0], _L)], vs[u],
                        mask=masks[u])
                return offv

            n_cand = comp8[0]
            nv_c = lax.shift_right_logical(n_cand + (_L - 1), 4)

            prefix = jnp.int32(0)
            kk = kk0
            for shift in (24, 16, 8, 0):
                clear_hist()

                def hist2(i, _c):
                    key = key_of(cand_v[pl.ds(i * _L, _L)])
                    valid = (i * _L + lanes) < n_cand
                    if shift == 24:
                        mask = valid
                    else:
                        mask = valid & (
                            lax.shift_right_logical(key, shift + 8)
                            == prefix)
                    byte = lax.shift_right_logical(key, shift) & 255
                    plsc.addupdate_scatter(
                        hist_v, [byte * _L + lanes], ones, mask=mask)
                    return 0

                lax.fori_loop(0, nv_c, hist2, 0)
                B, kk = suffix_and_search(kk)
                prefix = B if shift == 24 else (prefix * 256) | B

            bits = jnp.where(
                prefix < 0, prefix ^ jnp.int32(-0x80000000), ~prefix)
            thr = lax.bitcast_convert_type(bits, jnp.float32)
            thrv = jnp.broadcast_to(thr, (_L,))

            def scale(i):
                v = row_v[pl.ds(i * _L, _L)]
                row_v[pl.ds(i * _L, _L)] = jnp.where(v >= thrv, v * 10.0, v)

            unrolled(NV, 16, scale)
            pltpu.sync_copy(row_v, out_hbm.at[row_idx])
            return 0

        lax.fori_loop(0, rows_per_w, do_row, 0)

    return run


def kernel(scores, k):
    R, N = scores.shape
    info = plsc.get_sparse_core_info()
    NW = info.num_cores * info.num_subcores
    kk = jnp.full((_L,), jnp.asarray(k, jnp.int32) + 1, jnp.int32)
    return _build(R, N, NW)(scores, kk)
